# Initial kernel scaffold; baseline (speedup 1.0000x reference)
import jax
import jax.numpy as jnp
from jax.experimental import pallas as pl


def kernel(x, edge_index, batch, params):
    raise NotImplementedError("write your pallas kernel here")


# speedup vs baseline: 7.7785x; 7.7785x over previous
import functools

import jax
import jax.numpy as jnp
from jax import lax
from jax.experimental import pallas as pl
from jax.experimental.pallas import tpu as pltpu
from jax.experimental.pallas import tpu_sc as plsc

N = 50000
E = 800000
H = 64
G = 512
NB = 1000
NBLK = N // NB

NC, NS = 2, 16
NPAD = 50048
ROWS_PER_TILE = NPAD // NS

E1_PER_W = E // (NC * NS)
K1 = 1000

E2_PER_T = E // NS
K2 = 400

F1 = 8
FH = 32


def _sc_mesh():
    return plsc.VectorSubcoreMesh(core_axis_name="c", subcore_axis_name="s",
                                  num_cores=NC, num_subcores=NS)


def _sc_agg1(table, src, dst, zeros8):
    def body(tbl_ref, src_ref, dst_ref, z_ref, out_ref,
             sidx, didx, rows, acc, sem):
        core = lax.axis_index("c")
        sub = lax.axis_index("s")
        r0 = sub * ROWS_PER_TILE
        pltpu.sync_copy(z_ref.at[pl.ds(r0, ROWS_PER_TILE)],
                        acc.at[pl.ds(r0, ROWS_PER_TILE)])
        plsc.subcore_barrier()

        wid = core * NS + sub
        base = wid * E1_PER_W

        def chunk(j, carry):
            off = base + j * K1
            pltpu.sync_copy(src_ref.at[pl.ds(off, K1)], sidx)
            pltpu.sync_copy(dst_ref.at[pl.ds(off, K1)], didx)
            pltpu.async_copy(tbl_ref.at[sidx], rows, sem).wait()
            pltpu.sync_copy(rows, acc.at[didx], add=True)
            return carry

        lax.fori_loop(0, E1_PER_W // K1, chunk, 0)
        plsc.subcore_barrier()
        pltpu.sync_copy(acc.at[pl.ds(r0, ROWS_PER_TILE)],
                        out_ref.at[core, pl.ds(r0, ROWS_PER_TILE)])

    return pl.kernel(
        body,
        out_type=jax.ShapeDtypeStruct((NC, NPAD, F1), jnp.float32),
        mesh=_sc_mesh(),
        compiler_params=pltpu.CompilerParams(use_tc_tiling_on_sc=False),
        scratch_types=[
            pltpu.VMEM((K1,), jnp.int32),
            pltpu.VMEM((K1,), jnp.int32),
            pltpu.VMEM((K1, F1), jnp.float32),
            pltpu.VMEM_SHARED((NPAD, F1), jnp.float32),
            pltpu.SemaphoreType.DMA,
        ],
    )(table, src, dst, zeros8)


def _sc_aggh(ua, ub, src, dst, zeros32):
    def body(ua_ref, ub_ref, src_ref, dst_ref, z_ref, out_ref,
             sidx, didx, rows, acc, sem):
        core = lax.axis_index("c")
        sub = lax.axis_index("s")
        r0 = sub * ROWS_PER_TILE
        pltpu.sync_copy(z_ref.at[pl.ds(r0, ROWS_PER_TILE)],
                        acc.at[pl.ds(r0, ROWS_PER_TILE)])
        plsc.subcore_barrier()

        base = sub * E2_PER_T

        def chunk(j, carry):
            off = base + j * K2
            pltpu.sync_copy(src_ref.at[pl.ds(off, K2)], sidx)
            pltpu.sync_copy(dst_ref.at[pl.ds(off, K2)], didx)

            @pl.when(core == 0)
            def _():
                pltpu.async_copy(ua_ref.at[sidx], rows, sem).wait()

            @pl.when(core == 1)
            def _():
                pltpu.async_copy(ub_ref.at[sidx], rows, sem).wait()

            pltpu.sync_copy(rows, acc.at[didx], add=True)
            return carry

        lax.fori_loop(0, E2_PER_T // K2, chunk, 0)
        plsc.subcore_barrier()
        pltpu.sync_copy(acc.at[pl.ds(r0, ROWS_PER_TILE)],
                        out_ref.at[core, pl.ds(r0, ROWS_PER_TILE)])

    return pl.kernel(
        body,
        out_type=jax.ShapeDtypeStruct((NC, NPAD, FH), jnp.float32),
        mesh=_sc_mesh(),
        compiler_params=pltpu.CompilerParams(use_tc_tiling_on_sc=False),
        scratch_types=[
            pltpu.VMEM((K2,), jnp.int32),
            pltpu.VMEM((K2,), jnp.int32),
            pltpu.VMEM((K2, FH), jnp.float32),
            pltpu.VMEM_SHARED((NPAD, FH), jnp.float32),
            pltpu.SemaphoreType.DMA,
        ],
    )(ua, ub, src, dst, zeros32)


def _tc_layer1(x, p, w1, b1, w2, b2):
    def body(x_ref, p_ref, w1_ref, b1_ref, w2_ref, b2_ref,
             ua_ref, ub_ref, dcol_ref, sums_ref):
        i = pl.program_id(0)
        agg = p_ref[0] + p_ref[1]
        in1 = x_ref[...] + agg[:, 0:2]
        z = jnp.dot(in1, w1_ref[...], preferred_element_type=jnp.float32)
        z = jnp.maximum(z + b1_ref[...], 0.0)
        z = jnp.dot(z, w2_ref[...], preferred_element_type=jnp.float32)
        z = jnp.maximum(z + b2_ref[...], 0.0)
        ua_ref[...] = z[:, :FH]
        ub_ref[...] = z[:, FH:]
        dcol_ref[...] = jnp.broadcast_to(1.0 + agg[:, 2:3], (NB, F1))

        @pl.when(i == 0)
        def _():
            sums_ref[...] = jnp.zeros_like(sums_ref)

        sums_ref[0:1, :] += jnp.sum(z, axis=0, keepdims=True)
        sums_ref[1:2, :] += jnp.sum(z * z, axis=0, keepdims=True)

    return pl.pallas_call(
        body,
        grid=(NBLK,),
        in_specs=[
            pl.BlockSpec((NB, 2), lambda i: (i, 0)),
            pl.BlockSpec((NC, NB, F1), lambda i: (0, i, 0)),
            pl.BlockSpec((2, H), lambda i: (0, 0)),
            pl.BlockSpec((1, H), lambda i: (0, 0)),
            pl.BlockSpec((H, H), lambda i: (0, 0)),
            pl.BlockSpec((1, H), lambda i: (0, 0)),
        ],
        out_specs=[
            pl.BlockSpec((NB, FH), lambda i: (i, 0)),
            pl.BlockSpec((NB, FH), lambda i: (i, 0)),
            pl.BlockSpec((NB, F1), lambda i: (i, 0)),
            pl.BlockSpec((2, H), lambda i: (0, 0)),
        ],
        out_shape=[
            jax.ShapeDtypeStruct((N, FH), jnp.float32),
            jax.ShapeDtypeStruct((N, FH), jnp.float32),
            jax.ShapeDtypeStruct((N, F1), jnp.float32),
            jax.ShapeDtypeStruct((2, H), jnp.float32),
        ],
        compiler_params=pltpu.CompilerParams(
            dimension_semantics=("arbitrary",)),
    )(x, p, w1, b1.reshape(1, H), w2, b2.reshape(1, H))


def _tc_layer(ua, ub, aggu, dcol, a, c, w1, b1, w2, b2):
    def body(ua_ref, ub_ref, agg_ref, dcol_ref, ac_ref,
             w1_ref, b1_ref, w2_ref, b2_ref,
             oa_ref, ob_ref, sums_ref):
        i = pl.program_id(0)
        u = jnp.concatenate([ua_ref[...], ub_ref[...]], axis=1)
        agg = jnp.concatenate([agg_ref[0], agg_ref[1]], axis=1)
        inl = ac_ref[0:1, :] * (u + agg) + dcol_ref[...][:, 0:1] * ac_ref[1:2, :]
        z = jnp.dot(inl, w1_ref[...], preferred_element_type=jnp.float32)
        z = jnp.maximum(z + b1_ref[...], 0.0)
        z = jnp.dot(z, w2_ref[...], preferred_element_type=jnp.float32)
        z = jnp.maximum(z + b2_ref[...], 0.0)
        oa_ref[...] = z[:, :FH]
        ob_ref[...] = z[:, FH:]

        @pl.when(i == 0)
        def _():
            sums_ref[...] = jnp.zeros_like(sums_ref)

        sums_ref[0:1, :] += jnp.sum(z, axis=0, keepdims=True)
        sums_ref[1:2, :] += jnp.sum(z * z, axis=0, keepdims=True)

    ac = jnp.stack([a, c])
    return pl.pallas_call(
        body,
        grid=(NBLK,),
        in_specs=[
            pl.BlockSpec((NB, FH), lambda i: (i, 0)),
            pl.BlockSpec((NB, FH), lambda i: (i, 0)),
            pl.BlockSpec((NC, NB, FH), lambda i: (0, i, 0)),
            pl.BlockSpec((NB, F1), lambda i: (i, 0)),
            pl.BlockSpec((2, H), lambda i: (0, 0)),
            pl.BlockSpec((H, H), lambda i: (0, 0)),
            pl.BlockSpec((1, H), lambda i: (0, 0)),
            pl.BlockSpec((H, H), lambda i: (0, 0)),
            pl.BlockSpec((1, H), lambda i: (0, 0)),
        ],
        out_specs=[
            pl.BlockSpec((NB, FH), lambda i: (i, 0)),
            pl.BlockSpec((NB, FH), lambda i: (i, 0)),
            pl.BlockSpec((2, H), lambda i: (0, 0)),
        ],
        out_shape=[
            jax.ShapeDtypeStruct((N, FH), jnp.float32),
            jax.ShapeDtypeStruct((N, FH), jnp.float32),
            jax.ShapeDtypeStruct((2, H), jnp.float32),
        ],
        compiler_params=pltpu.CompilerParams(
            dimension_semantics=("arbitrary",)),
    )(ua, ub, aggu, dcol, ac, w1, b1.reshape(1, H), w2, b2.reshape(1, H))


def _tc_layer3_pool(ua, ub, aggu, dcol, a, c, w1, b1, w2, b2, batch3):
    def body(ua_ref, ub_ref, agg_ref, dcol_ref, ac_ref,
             w1_ref, b1_ref, w2_ref, b2_ref, bat_ref,
             pool_ref, sums_ref):
        i = pl.program_id(0)
        u = jnp.concatenate([ua_ref[...], ub_ref[...]], axis=1)
        agg = jnp.concatenate([agg_ref[0], agg_ref[1]], axis=1)
        inl = ac_ref[0:1, :] * (u + agg) + dcol_ref[...][:, 0:1] * ac_ref[1:2, :]
        z = jnp.dot(inl, w1_ref[...], preferred_element_type=jnp.float32)
        z = jnp.maximum(z + b1_ref[...], 0.0)
        z = jnp.dot(z, w2_ref[...], preferred_element_type=jnp.float32)
        z = jnp.maximum(z + b2_ref[...], 0.0)

        @pl.when(i == 0)
        def _():
            sums_ref[...] = jnp.zeros_like(sums_ref)
            pool_ref[...] = jnp.zeros_like(pool_ref)

        sums_ref[0:1, :] += jnp.sum(z, axis=0, keepdims=True)
        sums_ref[1:2, :] += jnp.sum(z * z, axis=0, keepdims=True)

        b = bat_ref[0, 0, :]
        gids = lax.broadcasted_iota(jnp.int32, (NB, G), 1)
        oh = jnp.where(b[:, None] == gids, 1.0, 0.0)
        z1 = jnp.concatenate([z, jnp.ones((NB, 1), jnp.float32)], axis=1)
        pool_ref[...] += lax.dot_general(
            oh, z1, (((0,), (0,)), ((), ())),
            preferred_element_type=jnp.float32)

    return pl.pallas_call(
        body,
        grid=(NBLK,),
        in_specs=[
            pl.BlockSpec((NB, FH), lambda i: (i, 0)),
            pl.BlockSpec((NB, FH), lambda i: (i, 0)),
            pl.BlockSpec((NC, NB, FH), lambda i: (0, i, 0)),
            pl.BlockSpec((NB, F1), lambda i: (i, 0)),
            pl.BlockSpec((2, H), lambda i: (0, 0)),
            pl.BlockSpec((H, H), lambda i: (0, 0)),
            pl.BlockSpec((1, H), lambda i: (0, 0)),
            pl.BlockSpec((H, H), lambda i: (0, 0)),
            pl.BlockSpec((1, H), lambda i: (0, 0)),
            pl.BlockSpec((1, 1, NB), lambda i: (i, 0, 0)),
        ],
        out_specs=[
            pl.BlockSpec((G, H + 1), lambda i: (0, 0)),
            pl.BlockSpec((2, H), lambda i: (0, 0)),
        ],
        out_shape=[
            jax.ShapeDtypeStruct((G, H + 1), jnp.float32),
            jax.ShapeDtypeStruct((2, H), jnp.float32),
        ],
        compiler_params=pltpu.CompilerParams(
            dimension_semantics=("arbitrary",)),
    )(ua, ub, aggu, dcol, jnp.stack([a, c]), w1, b1.reshape(1, H),
      w2, b2.reshape(1, H), batch3)


def _tc_head(pooled, sums3, gamma, beta, l1w, l1b, l2w, l2b):
    def body(pool_ref, s_ref, gb_ref, w1_ref, b1_ref, w2_ref, b2_ref, out_ref):
        mean = s_ref[0:1, :] * (1.0 / N)
        var = s_ref[1:2, :] * (1.0 / N) - mean * mean
        aa = gb_ref[0:1, :] * lax.rsqrt(var + 1e-5)
        cc = gb_ref[1:2, :] - mean * aa
        pu = pool_ref[...][:, :H]
        cnt = pool_ref[...][:, H:H + 1]
        p3 = aa * pu + cnt * cc
        hh = jnp.dot(p3, w1_ref[...], preferred_element_type=jnp.float32)
        hh = jnp.maximum(hh + b1_ref[...], 0.0)
        out = jnp.dot(hh, w2_ref[...], preferred_element_type=jnp.float32)
        out_ref[...] = out + b2_ref[...]

    return pl.pallas_call(
        body,
        out_shape=jax.ShapeDtypeStruct((G, 1), jnp.float32),
    )(pooled, sums3, jnp.stack([gamma, beta]), l1w, l1b.reshape(1, H),
      l2w, l2b.reshape(1, 1))


def _norm_coeffs(sums, gamma, beta):
    mean = sums[0] / N
    var = sums[1] / N - mean * mean
    a = gamma * jax.lax.rsqrt(var + 1e-5)
    return a, beta - mean * a


def kernel(x, edge_index, batch, params):
    src = edge_index[0]
    dst = edge_index[1]
    z8 = jnp.zeros((NPAD, F1), jnp.float32)
    z32 = jnp.zeros((NPAD, FH), jnp.float32)
    batch3 = batch.reshape(NBLK, 1, NB)

    table0 = jnp.concatenate(
        [x, jnp.ones((N, 1), jnp.float32), jnp.zeros((N, F1 - 3), jnp.float32)],
        axis=1)
    p0 = _sc_agg1(table0, src, dst, z8)
    c1p = params["conv1"]
    ua1, ub1, dcol, s1 = _tc_layer1(x, p0, c1p["W1"], c1p["b1"],
                                    c1p["W2"], c1p["b2"])
    a1, c1 = _norm_coeffs(s1, c1p["gamma"], c1p["beta"])

    p2 = params["convs"][0]
    agg1 = _sc_aggh(ua1, ub1, src, dst, z32)
    ua2, ub2, s2 = _tc_layer(ua1, ub1, agg1, dcol, a1, c1,
                             p2["W1"], p2["b1"], p2["W2"], p2["b2"])
    a2, c2 = _norm_coeffs(s2, p2["gamma"], p2["beta"])

    p3 = params["convs"][1]
    agg2 = _sc_aggh(ua2, ub2, src, dst, z32)
    pooled, s3 = _tc_layer3_pool(ua2, ub2, agg2, dcol, a2, c2,
                                 p3["W1"], p3["b1"], p3["W2"], p3["b2"],
                                 batch3)

    out = _tc_head(pooled, s3, p3["gamma"], p3["beta"],
                   params["lin1_W"], params["lin1_b"],
                   params["lin2_W"], params["lin2_b"])
    return out.reshape((G,))
